# baseline (speedup 1.0000x reference)
import functools

import jax
import jax.numpy as jnp
from jax import lax
from jax.experimental import pallas as pl
from jax.experimental.pallas import tpu as pltpu
from jax.experimental.pallas import tpu_sc as plsc

N = 10000
E = 320000
D = 128
NC = 2
NS = 16
NW = NC * NS
CH = 128
EPT = 10240
NCHUNK = EPT // CH
NHALF = 2
HC = NCHUNK // NHALF
HPAIR = HC // 2
NPAD = 10112
RPT = NPAD // NS


def _sc_segment_sum(nodes, send_r, recv_r, za, zc, ones):
    mesh = plsc.VectorSubcoreMesh(core_axis_name="c", subcore_axis_name="s")

    @functools.partial(
        pl.kernel,
        out_type=[
            jax.ShapeDtypeStruct((NC, NPAD, D), jnp.float32),
            jax.ShapeDtypeStruct((NC, NPAD), jnp.float32),
        ],
        mesh=mesh,
        scratch_types=[
            pltpu.VMEM((HC, CH), jnp.int32),
            pltpu.VMEM((HC, CH), jnp.int32),
            pltpu.VMEM((CH, D), jnp.float32),
            pltpu.VMEM((CH, D), jnp.float32),
            pltpu.VMEM((CH,), jnp.float32),
            pltpu.VMEM_SHARED((NPAD, D), jnp.float32),
            pltpu.VMEM_SHARED((NPAD,), jnp.float32),
            pltpu.SemaphoreType.DMA,
            pltpu.SemaphoreType.DMA,
            pltpu.SemaphoreType.DMA,
            pltpu.SemaphoreType.DMA,
        ],
    )
    def k(nodes_h, send_h, recv_h, za_h, zc_h, ones_h,
          a_out, c_out, sidx, ridx, rows0, rows1, onesv,
          a_sp, c_sp, sem0a, sem0b, sem1a, sem1b):
        c = lax.axis_index("c")
        s = lax.axis_index("s")
        w = c * NS + s

        pltpu.sync_copy(za_h.at[pl.ds(s * RPT, RPT)], a_sp.at[pl.ds(s * RPT, RPT)])

        @pl.when(s == 0)
        def _():
            pltpu.sync_copy(zc_h, c_sp)

        pltpu.sync_copy(ones_h, onesv)
        plsc.subcore_barrier()

        HH = CH // 2

        def gather(j, buf, sa, sb):
            pltpu.async_copy(
                nodes_h.at[sidx.at[j, pl.ds(0, HH)]], buf.at[pl.ds(0, HH)], sa)
            pltpu.async_copy(
                nodes_h.at[sidx.at[j, pl.ds(HH, HH)]], buf.at[pl.ds(HH, HH)], sb)

        def gwait(j, buf, sa, sb):
            pltpu.make_async_copy(
                nodes_h.at[sidx.at[j, pl.ds(0, HH)]],
                buf.at[pl.ds(0, HH)], sa).wait()
            pltpu.make_async_copy(
                nodes_h.at[sidx.at[j, pl.ds(HH, HH)]],
                buf.at[pl.ds(HH, HH)], sb).wait()

        def half(h, carry):
            pltpu.sync_copy(send_h.at[w, pl.ds(h * HC, HC)], sidx)
            pltpu.sync_copy(recv_h.at[w, pl.ds(h * HC, HC)], ridx)
            gather(0, rows0, sem0a, sem0b)

            def step(i, carry2):
                j0 = 2 * i
                j1 = j0 + 1
                gather(j1, rows1, sem1a, sem1b)
                gwait(j0, rows0, sem0a, sem0b)
                pltpu.sync_copy(rows0, a_sp.at[ridx.at[j0]], add=True)

                @pl.when(i < HPAIR - 1)
                def _():
                    gather(j1 + 1, rows0, sem0a, sem0b)

                pltpu.sync_copy(onesv, c_sp.at[ridx.at[j0]], add=True)
                gwait(j1, rows1, sem1a, sem1b)
                pltpu.sync_copy(rows1, a_sp.at[ridx.at[j1]], add=True)
                pltpu.sync_copy(onesv, c_sp.at[ridx.at[j1]], add=True)
                return carry2

            lax.fori_loop(0, HPAIR, step, 0)
            return carry

        lax.fori_loop(0, NHALF, half, 0)
        plsc.subcore_barrier()

        pltpu.sync_copy(a_sp.at[pl.ds(s * RPT, RPT)], a_out.at[c, pl.ds(s * RPT, RPT)])

        @pl.when(s == 0)
        def _():
            pltpu.sync_copy(c_sp, c_out.at[c])

    return k(nodes, send_r, recv_r, za, zc, ones)


def _tc_body(a_ref, c_ref, w_ref, b_ref, o_ref):
    a = a_ref[0] + a_ref[1]
    ct = c_ref[0] + c_ref[1]
    o_ref[...] = (
        jnp.dot(a, w_ref[...], preferred_element_type=jnp.float32)
        + ct * b_ref[...]
    )


def _tc_finish(a_parts, c_parts, W, b):
    BR = 400
    grid = (N // BR,)
    return pl.pallas_call(
        _tc_body,
        grid=grid,
        in_specs=[
            pl.BlockSpec((NC, BR, D), lambda i: (0, i, 0)),
            pl.BlockSpec((NC, BR, 1), lambda i: (0, i, 0)),
            pl.BlockSpec((D, D), lambda i: (0, 0)),
            pl.BlockSpec((1, D), lambda i: (0, 0)),
        ],
        out_specs=pl.BlockSpec((BR, D), lambda i: (i, 0)),
        out_shape=jax.ShapeDtypeStruct((N, D), jnp.float32),
    )(a_parts, c_parts.reshape(NC, NPAD, 1), W, b.reshape(1, D))


def kernel(nodes, senders, receivers, W_msg, b_msg):
    pad = NW * EPT - E
    send_r = jnp.concatenate(
        [senders, jnp.zeros((pad,), jnp.int32)]).reshape(NW, NCHUNK, CH)
    recv_r = jnp.concatenate(
        [receivers, jnp.full((pad,), N, jnp.int32)]).reshape(NW, NCHUNK, CH)
    za = jnp.zeros((NPAD, D), jnp.float32)
    zc = jnp.zeros((NPAD,), jnp.float32)
    ones = jnp.ones((CH,), jnp.float32)
    a_parts, c_parts = _sc_segment_sum(nodes, send_r, recv_r, za, zc, ones)
    return _tc_finish(a_parts, c_parts, W_msg, b_msg)


# speedup vs baseline: 1.0002x; 1.0002x over previous
import functools

import jax
import jax.numpy as jnp
from jax import lax
from jax.experimental import pallas as pl
from jax.experimental.pallas import tpu as pltpu
from jax.experimental.pallas import tpu_sc as plsc

N = 10000
E = 320000
D = 128
NC = 2
NS = 16
NW = NC * NS
CH = 128
EPT = 10240
NCHUNK = EPT // CH
NHALF = 2
HC = NCHUNK // NHALF
HPAIR = HC // 2
NPAD = 10240
RPT = NPAD // NS


def _sc_segment_sum(nodes, send_r, recv_r, za, zc, ones):
    mesh = plsc.VectorSubcoreMesh(core_axis_name="c", subcore_axis_name="s")

    @functools.partial(
        pl.kernel,
        out_type=[
            jax.ShapeDtypeStruct((NC, NPAD, D), jnp.float32),
            jax.ShapeDtypeStruct((NC, NPAD), jnp.float32),
        ],
        mesh=mesh,
        scratch_types=[
            pltpu.VMEM((HC, CH), jnp.int32),
            pltpu.VMEM((HC, CH), jnp.int32),
            pltpu.VMEM((CH, D), jnp.float32),
            pltpu.VMEM((CH, D), jnp.float32),
            pltpu.VMEM((CH,), jnp.float32),
            pltpu.VMEM_SHARED((NPAD, D), jnp.float32),
            pltpu.VMEM_SHARED((NPAD,), jnp.float32),
            pltpu.SemaphoreType.DMA,
            pltpu.SemaphoreType.DMA,
        ],
    )
    def k(nodes_h, send_h, recv_h, za_h, zc_h, ones_h,
          a_out, c_out, sidx, ridx, rows0, rows1, onesv, a_sp, c_sp,
          sem0, sem1):
        c = lax.axis_index("c")
        s = lax.axis_index("s")
        w = c * NS + s

        pltpu.sync_copy(za_h.at[pl.ds(s * RPT, RPT)], a_sp.at[pl.ds(s * RPT, RPT)])

        @pl.when(s == 0)
        def _():
            pltpu.sync_copy(zc_h, c_sp)

        pltpu.sync_copy(ones_h, onesv)
        plsc.subcore_barrier()

        def half(h, carry):
            pltpu.sync_copy(send_h.at[w, pl.ds(h * HC, HC)], sidx)
            pltpu.sync_copy(recv_h.at[w, pl.ds(h * HC, HC)], ridx)
            pltpu.async_copy(nodes_h.at[sidx.at[0]], rows0, sem0)

            def step(i, carry2):
                j0 = 2 * i
                j1 = j0 + 1
                pltpu.async_copy(nodes_h.at[sidx.at[j1]], rows1, sem1)
                pltpu.make_async_copy(nodes_h.at[sidx.at[j0]], rows0, sem0).wait()
                pltpu.sync_copy(rows0, a_sp.at[ridx.at[j0]], add=True)

                @pl.when(i < HPAIR - 1)
                def _():
                    pltpu.async_copy(nodes_h.at[sidx.at[j1 + 1]], rows0, sem0)

                pltpu.sync_copy(onesv, c_sp.at[ridx.at[j0]], add=True)
                pltpu.make_async_copy(nodes_h.at[sidx.at[j1]], rows1, sem1).wait()
                pltpu.sync_copy(rows1, a_sp.at[ridx.at[j1]], add=True)
                pltpu.sync_copy(onesv, c_sp.at[ridx.at[j1]], add=True)
                return carry2

            lax.fori_loop(0, HPAIR, step, 0)
            return carry

        lax.fori_loop(0, NHALF, half, 0)
        plsc.subcore_barrier()

        pltpu.sync_copy(a_sp.at[pl.ds(s * RPT, RPT)], a_out.at[c, pl.ds(s * RPT, RPT)])

        @pl.when(s == 0)
        def _():
            pltpu.sync_copy(c_sp, c_out.at[c])

    return k(nodes, send_r, recv_r, za, zc, ones)


def _tc_body(a_ref, c_ref, w_ref, b_ref, o_ref):
    a = a_ref[0] + a_ref[1]
    ct = c_ref[0] + c_ref[1]
    o_ref[...] = (
        jnp.dot(a, w_ref[...], preferred_element_type=jnp.float32)
        + ct * b_ref[...]
    )


def _tc_finish(a_parts, c_parts, W, b):
    BR = 400
    grid = (N // BR,)
    return pl.pallas_call(
        _tc_body,
        grid=grid,
        in_specs=[
            pl.BlockSpec((NC, BR, D), lambda i: (0, i, 0)),
            pl.BlockSpec((NC, BR, 1), lambda i: (0, i, 0)),
            pl.BlockSpec((D, D), lambda i: (0, 0)),
            pl.BlockSpec((1, D), lambda i: (0, 0)),
        ],
        out_specs=pl.BlockSpec((BR, D), lambda i: (i, 0)),
        out_shape=jax.ShapeDtypeStruct((N, D), jnp.float32),
    )(a_parts, c_parts.reshape(NC, NPAD, 1), W, b.reshape(1, D))


def kernel(nodes, senders, receivers, W_msg, b_msg):
    pad = NW * EPT - E
    send_r = jnp.concatenate(
        [senders, jnp.zeros((pad,), jnp.int32)]).reshape(NW, NCHUNK, CH)
    recv_r = jnp.concatenate(
        [receivers, jnp.full((pad,), N, jnp.int32)]).reshape(NW, NCHUNK, CH)
    za = jnp.zeros((NPAD, D), jnp.float32)
    zc = jnp.zeros((NPAD,), jnp.float32)
    ones = jnp.ones((CH,), jnp.float32)
    a_parts, c_parts = _sc_segment_sum(nodes, send_r, recv_r, za, zc, ones)
    return _tc_finish(a_parts, c_parts, W_msg, b_msg)


# speedup vs baseline: 1.0003x; 1.0002x over previous
import functools

import jax
import jax.numpy as jnp
from jax import lax
from jax.experimental import pallas as pl
from jax.experimental.pallas import tpu as pltpu
from jax.experimental.pallas import tpu_sc as plsc

N = 10000
E = 320000
D = 128
NC = 2
NS = 16
NW = NC * NS
CH = 128
EPT = 10240
NCHUNK = EPT // CH
NHALF = 2
HC = NCHUNK // NHALF
HPAIR = HC // 2
NPAD = 10240
RPT = NPAD // NS


def _sc_segment_sum(nodes, send_r, recv_r, za, zc, ones):
    mesh = plsc.VectorSubcoreMesh(core_axis_name="c", subcore_axis_name="s")

    @functools.partial(
        pl.kernel,
        out_type=[
            jax.ShapeDtypeStruct((NC, NPAD, D), jnp.float32),
            jax.ShapeDtypeStruct((NC, NPAD), jnp.float32),
        ],
        mesh=mesh,
        scratch_types=[
            pltpu.VMEM((HC, CH), jnp.int32),
            pltpu.VMEM((HC, CH), jnp.int32),
            pltpu.VMEM((CH, D), jnp.float32),
            pltpu.VMEM((CH, D), jnp.float32),
            pltpu.VMEM((CH,), jnp.float32),
            pltpu.VMEM_SHARED((NPAD, D), jnp.float32),
            pltpu.VMEM_SHARED((NPAD,), jnp.float32),
            pltpu.SemaphoreType.DMA,
            pltpu.SemaphoreType.DMA,
            pltpu.SemaphoreType.DMA,
        ],
    )
    def k(nodes_h, send_h, recv_h, za_h, zc_h, ones_h,
          a_out, c_out, sidx, ridx, rows0, rows1, onesv, a_sp, c_sp,
          sem0, sem1, semc):
        c = lax.axis_index("c")
        s = lax.axis_index("s")
        w = c * NS + s

        pltpu.sync_copy(za_h.at[pl.ds(s * RPT, RPT)], a_sp.at[pl.ds(s * RPT, RPT)])

        @pl.when(s == 0)
        def _():
            pltpu.sync_copy(zc_h, c_sp)

        pltpu.sync_copy(ones_h, onesv)
        plsc.subcore_barrier()

        def half(h, carry):
            pltpu.sync_copy(send_h.at[w, pl.ds(h * HC, HC)], sidx)
            pltpu.sync_copy(recv_h.at[w, pl.ds(h * HC, HC)], ridx)
            pltpu.async_copy(nodes_h.at[sidx.at[0]], rows0, sem0)

            def step(i, carry2):
                j0 = 2 * i
                j1 = j0 + 1
                pltpu.async_copy(nodes_h.at[sidx.at[j1]], rows1, sem1)
                pltpu.make_async_copy(nodes_h.at[sidx.at[j0]], rows0, sem0).wait()
                pltpu.sync_copy(rows0, a_sp.at[ridx.at[j0]], add=True)

                @pl.when(i < HPAIR - 1)
                def _():
                    pltpu.async_copy(nodes_h.at[sidx.at[j1 + 1]], rows0, sem0)

                pltpu.async_copy(onesv, c_sp.at[ridx.at[j0]], semc, add=True)
                pltpu.make_async_copy(nodes_h.at[sidx.at[j1]], rows1, sem1).wait()
                pltpu.sync_copy(rows1, a_sp.at[ridx.at[j1]], add=True)
                pltpu.async_copy(onesv, c_sp.at[ridx.at[j1]], semc, add=True)
                return carry2

            lax.fori_loop(0, HPAIR, step, 0)
            return carry

        lax.fori_loop(0, NHALF, half, 0)

        def drain(i, carry):
            pltpu.make_async_copy(onesv, c_sp.at[ridx.at[0]], semc).wait()
            return carry

        lax.fori_loop(0, NCHUNK, drain, 0)
        plsc.subcore_barrier()

        pltpu.sync_copy(a_sp.at[pl.ds(s * RPT, RPT)], a_out.at[c, pl.ds(s * RPT, RPT)])

        @pl.when(s == 0)
        def _():
            pltpu.sync_copy(c_sp, c_out.at[c])

    return k(nodes, send_r, recv_r, za, zc, ones)


def _tc_body(a_ref, c_ref, w_ref, b_ref, o_ref):
    a = a_ref[0] + a_ref[1]
    ct = c_ref[0] + c_ref[1]
    o_ref[...] = (
        jnp.dot(a, w_ref[...], preferred_element_type=jnp.float32)
        + ct * b_ref[...]
    )


def _tc_finish(a_parts, c_parts, W, b):
    BR = 400
    grid = (N // BR,)
    return pl.pallas_call(
        _tc_body,
        grid=grid,
        in_specs=[
            pl.BlockSpec((NC, BR, D), lambda i: (0, i, 0)),
            pl.BlockSpec((NC, BR, 1), lambda i: (0, i, 0)),
            pl.BlockSpec((D, D), lambda i: (0, 0)),
            pl.BlockSpec((1, D), lambda i: (0, 0)),
        ],
        out_specs=pl.BlockSpec((BR, D), lambda i: (i, 0)),
        out_shape=jax.ShapeDtypeStruct((N, D), jnp.float32),
    )(a_parts, c_parts.reshape(NC, NPAD, 1), W, b.reshape(1, D))


def kernel(nodes, senders, receivers, W_msg, b_msg):
    pad = NW * EPT - E
    send_r = jnp.concatenate(
        [senders, jnp.zeros((pad,), jnp.int32)]).reshape(NW, NCHUNK, CH)
    recv_r = jnp.concatenate(
        [receivers, jnp.full((pad,), N, jnp.int32)]).reshape(NW, NCHUNK, CH)
    za = jnp.zeros((NPAD, D), jnp.float32)
    zc = jnp.zeros((NPAD,), jnp.float32)
    ones = jnp.ones((CH,), jnp.float32)
    a_parts, c_parts = _sc_segment_sum(nodes, send_r, recv_r, za, zc, ones)
    return _tc_finish(a_parts, c_parts, W_msg, b_msg)
